# baseline (speedup 1.0000x reference)
import functools

import jax
import jax.numpy as jnp
from jax import lax
from jax.experimental import pallas as pl
from jax.experimental.pallas import tpu as pltpu
from jax.experimental.pallas import tpu_sc as plsc

BATCH = 16384
TEXT_DIM = 128
FINAL_DIM = 64
HIDDEN = (TEXT_DIM + 1) // 2
NUM_USERS = 1000000

NUM_CORES = 2
NUM_SUBCORES = 16
NW = NUM_CORES * NUM_SUBCORES
BPW = BATCH // NW
CHUNK = 128
NCH = BPW // CHUNK


def _sc_gather_128(ids2d, table, d_model):
  mesh = plsc.VectorSubcoreMesh(core_axis_name="c", subcore_axis_name="s")

  @functools.partial(
      pl.kernel,
      out_type=jax.ShapeDtypeStruct((BATCH, d_model), jnp.float32),
      mesh=mesh,
      scratch_types=[
          pltpu.VMEM((NCH, CHUNK), jnp.int32),
          pltpu.VMEM((BPW, d_model), jnp.float32),
          pltpu.SemaphoreType.DMA,
      ],
  )
  def k(ids_hbm, tab_hbm, out_hbm, idx, rows, sem):
    wid = lax.axis_index("s") * NUM_CORES + lax.axis_index("c")
    row0 = wid * NCH
    pltpu.sync_copy(ids_hbm.at[pl.ds(row0, NCH)], idx)
    heads = []
    for j in range(NCH):
      heads.append(pltpu.async_copy(
          tab_hbm.at[idx.at[j]], rows.at[pl.ds(j * CHUNK, CHUNK)], sem))
    for h in heads:
      h.wait()
    pltpu.sync_copy(rows, out_hbm.at[pl.ds(wid * BPW, BPW)])

  return k(ids2d, table)


def _mlp_body(x_ref, p_ref, up_ref, par_ref, w1m_ref, w1l_ref, b1_ref,
              w2t_ref, b2_ref, o_ref, u_ref):
  h = jnp.dot(x_ref[...], w1m_ref[...], preferred_element_type=jnp.float32)
  h = h + p_ref[...] * w1l_ref[...] + b1_ref[...]
  h = jnp.maximum(h, 0.0)
  o_ref[...] = (
      jnp.dot(h, w2t_ref[...], preferred_element_type=jnp.float32)
      + b2_ref[...])
  up = up_ref[...]
  u_ref[...] = jnp.where(par_ref[...] > 0, up[:, FINAL_DIM:],
                         up[:, :FINAL_DIM])


def _mlp_and_select(text_vecs, prices_col, upairs, parity_col,
                    w1m, w1l, b1r, w2t, b2r, block_m=2048):
  grid = (BATCH // block_m,)
  return pl.pallas_call(
      _mlp_body,
      grid=grid,
      in_specs=[
          pl.BlockSpec((block_m, TEXT_DIM), lambda i: (i, 0)),
          pl.BlockSpec((block_m, 1), lambda i: (i, 0)),
          pl.BlockSpec((block_m, TEXT_DIM), lambda i: (i, 0)),
          pl.BlockSpec((block_m, 1), lambda i: (i, 0)),
          pl.BlockSpec((TEXT_DIM, HIDDEN), lambda i: (0, 0)),
          pl.BlockSpec((1, HIDDEN), lambda i: (0, 0)),
          pl.BlockSpec((1, HIDDEN), lambda i: (0, 0)),
          pl.BlockSpec((HIDDEN, FINAL_DIM), lambda i: (0, 0)),
          pl.BlockSpec((1, FINAL_DIM), lambda i: (0, 0)),
      ],
      out_specs=[
          pl.BlockSpec((block_m, FINAL_DIM), lambda i: (i, 0)),
          pl.BlockSpec((block_m, FINAL_DIM), lambda i: (i, 0)),
      ],
      out_shape=[
          jax.ShapeDtypeStruct((BATCH, FINAL_DIM), jnp.float32),
          jax.ShapeDtypeStruct((BATCH, FINAL_DIM), jnp.float32),
      ],
  )(text_vecs, prices_col, upairs, parity_col, w1m, w1l, b1r, w2t, b2r)


def kernel(user_ids, item_ids, item_prices, user_table, item_text_table,
           W1, b1, W2, b2):
  uids = user_ids.astype(jnp.int32)
  iids2 = item_ids.astype(jnp.int32).reshape(BATCH // CHUNK, CHUNK)
  upair_ids2 = (uids // 2).reshape(BATCH // CHUNK, CHUNK)
  parity_col = (uids % 2).reshape(BATCH, 1)
  text_vecs = _sc_gather_128(iids2, item_text_table, TEXT_DIM)
  utab_pairs = user_table.reshape(NUM_USERS // 2, 2 * FINAL_DIM)
  upairs = _sc_gather_128(upair_ids2, utab_pairs, 2 * FINAL_DIM)
  w1m = W1[:, :TEXT_DIM].T
  w1l = W1[:, TEXT_DIM:].T
  item_vec, user_vec = _mlp_and_select(
      text_vecs, item_prices.reshape(BATCH, 1), upairs, parity_col,
      w1m, w1l, b1.reshape(1, HIDDEN), W2.T, b2.reshape(1, FINAL_DIM))
  return user_vec, item_vec


# speedup vs baseline: 2.4565x; 2.4565x over previous
import functools

import jax
import jax.numpy as jnp
from jax import lax
from jax.experimental import pallas as pl
from jax.experimental.pallas import tpu as pltpu
from jax.experimental.pallas import tpu_sc as plsc

BATCH = 16384
TEXT_DIM = 128
FINAL_DIM = 64
HIDDEN = (TEXT_DIM + 1) // 2
NUM_USERS = 1000000
HALF_USERS = NUM_USERS // 2

NUM_CORES = 2
NUM_SUBCORES = 16
NW = NUM_CORES * NUM_SUBCORES
BPW = BATCH // NW
CHUNK = 128
NCH = BPW // CHUNK


def _sc_item_gather(ids2d, table):
  mesh = plsc.VectorSubcoreMesh(core_axis_name="c", subcore_axis_name="s")

  @functools.partial(
      pl.kernel,
      out_type=jax.ShapeDtypeStruct((BATCH, TEXT_DIM), jnp.float32),
      mesh=mesh,
      scratch_types=[
          pltpu.VMEM((NCH, CHUNK), jnp.int32),
          pltpu.VMEM((BPW, TEXT_DIM), jnp.float32),
          pltpu.SemaphoreType.DMA,
      ],
  )
  def k(ids_hbm, tab_hbm, out_hbm, idx, rows, sem):
    wid = lax.axis_index("s") * NUM_CORES + lax.axis_index("c")
    row0 = wid * NCH
    pltpu.sync_copy(ids_hbm.at[pl.ds(row0, NCH)], idx)
    heads = []
    for j in range(NCH):
      heads.append(pltpu.async_copy(
          tab_hbm.at[idx.at[j]], rows.at[pl.ds(j * CHUNK, CHUNK)], sem))
    for h in heads:
      h.wait()
    pltpu.sync_copy(rows, out_hbm.at[pl.ds(wid * BPW, BPW)])

  return k(ids2d, table)


def _sc_user_gather(uids2d, utab3):
  mesh = plsc.VectorSubcoreMesh(core_axis_name="c", subcore_axis_name="s")

  @functools.partial(
      pl.kernel,
      out_type=jax.ShapeDtypeStruct((BATCH, FINAL_DIM), jnp.float32),
      mesh=mesh,
      scratch_types=[
          pltpu.VMEM((1, BPW), jnp.int32),
          pltpu.VMEM((BPW, FINAL_DIM), jnp.float32),
          pltpu.SemaphoreType.DMA,
      ],
  )
  def k(uids_hbm, utab_hbm, uout_hbm, uidx_v, urows, usem):
    wid = lax.axis_index("s") * NUM_CORES + lax.axis_index("c")
    pltpu.sync_copy(uids_hbm.at[pl.ds(wid, 1)], uidx_v)

    def row_dma_group(g, carry):
      v = uidx_v[0, pl.ds(g * 16, 16)]
      base16 = g * 16
      for lane in range(16):
        r = v[lane]
        h = r // HALF_USERS
        rr = r % HALF_USERS
        pltpu.async_copy(
            utab_hbm.at[h].at[pl.ds(rr, 1)],
            urows.at[pl.ds(base16 + lane, 1)], usem)
      return carry

    lax.fori_loop(0, BPW // 16, row_dma_group, 0)
    pltpu.make_async_copy(
        utab_hbm.at[0].at[pl.ds(0, BPW)], urows, usem).wait()
    pltpu.sync_copy(urows, uout_hbm.at[pl.ds(wid * BPW, BPW)])

  return k(uids2d, utab3)


def _mlp_body(x_ref, p_ref, w1m_ref, w1l_ref, b1_ref, w2t_ref, b2_ref, o_ref):
  h = jnp.dot(x_ref[...], w1m_ref[...], preferred_element_type=jnp.float32)
  h = h + p_ref[...] * w1l_ref[...] + b1_ref[...]
  h = jnp.maximum(h, 0.0)
  o_ref[...] = (
      jnp.dot(h, w2t_ref[...], preferred_element_type=jnp.float32)
      + b2_ref[...])


def _mlp(text_vecs, prices_col, w1m, w1l, b1r, w2t, b2r, block_m=2048):
  grid = (BATCH // block_m,)
  return pl.pallas_call(
      _mlp_body,
      grid=grid,
      in_specs=[
          pl.BlockSpec((block_m, TEXT_DIM), lambda i: (i, 0)),
          pl.BlockSpec((block_m, 1), lambda i: (i, 0)),
          pl.BlockSpec((TEXT_DIM, HIDDEN), lambda i: (0, 0)),
          pl.BlockSpec((1, HIDDEN), lambda i: (0, 0)),
          pl.BlockSpec((1, HIDDEN), lambda i: (0, 0)),
          pl.BlockSpec((HIDDEN, FINAL_DIM), lambda i: (0, 0)),
          pl.BlockSpec((1, FINAL_DIM), lambda i: (0, 0)),
      ],
      out_specs=pl.BlockSpec((block_m, FINAL_DIM), lambda i: (i, 0)),
      out_shape=jax.ShapeDtypeStruct((BATCH, FINAL_DIM), jnp.float32),
  )(text_vecs, prices_col, w1m, w1l, b1r, w2t, b2r)


def kernel(user_ids, item_ids, item_prices, user_table, item_text_table,
           W1, b1, W2, b2):
  uids2 = user_ids.astype(jnp.int32).reshape(NW, BPW)
  iids2 = item_ids.astype(jnp.int32).reshape(BATCH // CHUNK, CHUNK)
  text_vecs = _sc_item_gather(iids2, item_text_table)
  utab3 = user_table.reshape(2, HALF_USERS, FINAL_DIM)
  user_vec = _sc_user_gather(uids2, utab3)
  w1m = W1[:, :TEXT_DIM].T
  w1l = W1[:, TEXT_DIM:].T
  item_vec = _mlp(text_vecs, item_prices.reshape(BATCH, 1), w1m, w1l,
                  b1.reshape(1, HIDDEN), W2.T, b2.reshape(1, FINAL_DIM))
  return user_vec, item_vec


# speedup vs baseline: 2.5338x; 1.0315x over previous
import functools

import jax
import jax.numpy as jnp
from jax import lax
from jax.experimental import pallas as pl
from jax.experimental.pallas import tpu as pltpu
from jax.experimental.pallas import tpu_sc as plsc

BATCH = 16384
TEXT_DIM = 128
FINAL_DIM = 64
HIDDEN = (TEXT_DIM + 1) // 2
NUM_USERS = 1000000
HALF_USERS = NUM_USERS // 2

NUM_CORES = 2
NUM_SUBCORES = 16
NW = NUM_CORES * NUM_SUBCORES
BPW = BATCH // NW
CHUNK = 128
NCH = BPW // CHUNK


def _sc_item_gather(ids2d, table):
  mesh = plsc.VectorSubcoreMesh(core_axis_name="c", subcore_axis_name="s")

  @functools.partial(
      pl.kernel,
      out_type=jax.ShapeDtypeStruct((BATCH, TEXT_DIM), jnp.float32),
      mesh=mesh,
      scratch_types=[
          pltpu.VMEM((NCH, CHUNK), jnp.int32),
          pltpu.VMEM((BPW, TEXT_DIM), jnp.float32),
          pltpu.SemaphoreType.DMA,
      ],
  )
  def k(ids_hbm, tab_hbm, out_hbm, idx, rows, sem):
    wid = lax.axis_index("s") * NUM_CORES + lax.axis_index("c")
    row0 = wid * NCH
    pltpu.sync_copy(ids_hbm.at[pl.ds(row0, NCH)], idx)
    heads = []
    for j in range(NCH):
      heads.append(pltpu.async_copy(
          tab_hbm.at[idx.at[j]], rows.at[pl.ds(j * CHUNK, CHUNK)], sem))
    for h in heads:
      h.wait()
    pltpu.sync_copy(rows, out_hbm.at[pl.ds(wid * BPW, BPW)])

  return k(ids2d, table)


def _sc_user_gather(uids2d, utab3):
  mesh = plsc.VectorSubcoreMesh(core_axis_name="c", subcore_axis_name="s")

  @functools.partial(
      pl.kernel,
      out_type=jax.ShapeDtypeStruct((BATCH, FINAL_DIM), jnp.float32),
      mesh=mesh,
      scratch_types=[
          pltpu.VMEM((1, BPW), jnp.int32),
          pltpu.VMEM((BPW, FINAL_DIM), jnp.float32),
          pltpu.SemaphoreType.DMA,
      ],
  )
  def k(uids_hbm, utab_hbm, uout_hbm, uidx_v, urows, usem):
    wid = lax.axis_index("s") * NUM_CORES + lax.axis_index("c")
    pltpu.sync_copy(uids_hbm.at[pl.ds(wid, 1)], uidx_v)

    def row_dma_group(g, carry):
      v = uidx_v[0, pl.ds(g * 16, 16)]
      base16 = g * 16
      for lane in range(16):
        r = v[lane]
        h = r // HALF_USERS
        rr = r % HALF_USERS
        pltpu.async_copy(
            utab_hbm.at[h].at[pl.ds(rr, 1)],
            urows.at[pl.ds(base16 + lane, 1)], usem)
      return carry

    lax.fori_loop(0, BPW // 16, row_dma_group, 0)
    pltpu.make_async_copy(
        utab_hbm.at[0].at[pl.ds(0, BPW)], urows, usem).wait()
    pltpu.sync_copy(urows, uout_hbm.at[pl.ds(wid * BPW, BPW)])

  return k(uids2d, utab3)


def _mlp_body(x_ref, p_ref, w1m_ref, w1l_ref, b1_ref, w2t_ref, b2_ref, o_ref):
  h = jnp.dot(x_ref[...], w1m_ref[...], preferred_element_type=jnp.float32)
  h = h + p_ref[...] * w1l_ref[...] + b1_ref[...]
  h = jnp.maximum(h, 0.0)
  out = (jnp.dot(h, w2t_ref[...], preferred_element_type=jnp.float32)
         + b2_ref[...])
  o_ref[...] = out.T


def _mlp(text_vecs, prices_col, w1m, w1l, b1r, w2t, b2r, block_m=2048):
  grid = (BATCH // block_m,)
  return pl.pallas_call(
      _mlp_body,
      grid=grid,
      in_specs=[
          pl.BlockSpec((block_m, TEXT_DIM), lambda i: (i, 0)),
          pl.BlockSpec((block_m, 1), lambda i: (i, 0)),
          pl.BlockSpec((TEXT_DIM, HIDDEN), lambda i: (0, 0)),
          pl.BlockSpec((1, HIDDEN), lambda i: (0, 0)),
          pl.BlockSpec((1, HIDDEN), lambda i: (0, 0)),
          pl.BlockSpec((HIDDEN, FINAL_DIM), lambda i: (0, 0)),
          pl.BlockSpec((1, FINAL_DIM), lambda i: (0, 0)),
      ],
      out_specs=pl.BlockSpec((FINAL_DIM, block_m), lambda i: (0, i)),
      out_shape=jax.ShapeDtypeStruct((FINAL_DIM, BATCH), jnp.float32),
  )(text_vecs, prices_col, w1m, w1l, b1r, w2t, b2r)


def kernel(user_ids, item_ids, item_prices, user_table, item_text_table,
           W1, b1, W2, b2):
  uids2 = user_ids.astype(jnp.int32).reshape(NW, BPW)
  iids2 = item_ids.astype(jnp.int32).reshape(BATCH // CHUNK, CHUNK)
  text_vecs = _sc_item_gather(iids2, item_text_table)
  utab3 = user_table.reshape(2, HALF_USERS, FINAL_DIM)
  user_vec = _sc_user_gather(uids2, utab3)
  w1m = W1[:, :TEXT_DIM].T
  w1l = W1[:, TEXT_DIM:].T
  item_vec_t = _mlp(text_vecs, item_prices.reshape(BATCH, 1), w1m, w1l,
                    b1.reshape(1, HIDDEN), W2.T, b2.reshape(1, FINAL_DIM))
  return user_vec, item_vec_t.T
